# baseline (speedup 1.0000x reference)
import functools
import jax
import jax.numpy as jnp
from jax import lax
from jax.experimental import pallas as pl
from jax.experimental.pallas import tpu as pltpu
from jax.experimental.pallas import tpu_sc as plsc

BATCH = 2048
IN_DIM = 4096
OUT_DIM = 8192
L = 16
NW = 32
ROWS_PER_W = BATCH // NW
R = 4
NBLK = ROWS_PER_W // R
NGC = OUT_DIM // L
SIGN = jnp.int32(-2147483648)
MASK = jnp.int32(0x7FFFFFFF)


def _sc_body(x_hbm, ll_hbm, ia_hbm, ib_hbm, out_hbm,
             pia_v, pib_v, xblk_v, oblk_v):
    wid = lax.axis_index("s") * 2 + lax.axis_index("c")
    row_base = wid * ROWS_PER_W

    pltpu.sync_copy(ia_hbm, pia_v)
    pltpu.sync_copy(ib_hbm, pib_v)
    pltpu.sync_copy(ll_hbm, oblk_v.at[pl.ds(0, 2 * OUT_DIM)])

    @pl.loop(0, NGC)
    def init_consts(gc):
        s = pl.ds(gc * L, L)
        j2 = (gc * (2 * L)) + 2 * lax.iota(jnp.int32, 16)
        la = plsc.load_gather(oblk_v, [j2])
        lb = plsc.load_gather(oblk_v, [j2 + 1])
        pia_v[s] = pia_v[s] | jnp.where(la > 0.0, SIGN, 0)
        pib_v[s] = pib_v[s] | jnp.where(lb > 0.0, SIGN, 0)

    @pl.loop(0, NBLK)
    def do_block(blk):
        row0 = row_base + blk * R
        pltpu.sync_copy(x_hbm.at[pl.ds(row0 * IN_DIM, R * IN_DIM)], xblk_v)
        plsc.subcore_barrier()

        @pl.loop(0, NGC, unroll=8)
        def do_chunk(gc):
            s = pl.ds(gc * L, L)
            pia = pia_v[s]
            pib = pib_v[s]
            ca = lax.shift_right_logical(pia, 31).astype(jnp.float32)
            cb = lax.shift_right_logical(pib, 31).astype(jnp.float32)
            sa = 1.0 - 2.0 * ca
            sb = 1.0 - 2.0 * cb
            ia = pia & MASK
            ib = pib & MASK
            for r in range(R):
                a = plsc.load_gather(xblk_v, [ia + (r * IN_DIM)])
                bb = plsc.load_gather(xblk_v, [ib + (r * IN_DIM)])
                oblk_v[pl.ds(r * OUT_DIM + gc * L, L)] = (
                    (ca + sa * a) * (cb + sb * bb))

        plsc.subcore_barrier()
        pltpu.sync_copy(oblk_v, out_hbm.at[pl.ds(row0 * OUT_DIM, R * OUT_DIM)])


@jax.jit
def kernel(x, negation_logits, idx_a, idx_b):
    mesh = plsc.VectorSubcoreMesh(core_axis_name="c", subcore_axis_name="s")
    f = pl.kernel(
        _sc_body,
        out_type=jax.ShapeDtypeStruct((BATCH * OUT_DIM,), jnp.float32),
        mesh=mesh,
        compiler_params=pltpu.CompilerParams(needs_layout_passes=False),
        scratch_types=[
            pltpu.VMEM((OUT_DIM,), jnp.int32),
            pltpu.VMEM((OUT_DIM,), jnp.int32),
            pltpu.VMEM((R * IN_DIM,), jnp.float32),
            pltpu.VMEM((R * OUT_DIM,), jnp.float32),
        ],
    )
    out = f(x.reshape(-1), negation_logits.reshape(-1), idx_a, idx_b)
    return out.reshape(BATCH, OUT_DIM)


# speedup vs baseline: 1.4632x; 1.4632x over previous
import jax
import jax.numpy as jnp
from jax import lax
from jax.experimental import pallas as pl
from jax.experimental.pallas import tpu as pltpu
from jax.experimental.pallas import tpu_sc as plsc

BATCH = 2048
IN_DIM = 4096
OUT_DIM = 8192
L = 16
NW = 32
ROWS_PER_W = BATCH // NW
R = 4
NBLK = ROWS_PER_W // R
NGC = OUT_DIM // L
SIGN = jnp.int32(-2147483648)
MASK = jnp.int32(0x7FFFFFFF)


def _zero_dep(vec_i32):
    return jnp.sum(vec_i32) & 0


def _sc_body(x_hbm, ll_hbm, ia_hbm, ib_hbm, out_hbm,
             pia_v, pib_v, lg_v, xblk_v, oblk_v):
    wid = lax.axis_index("s") * 2 + lax.axis_index("c")
    row_base = wid * ROWS_PER_W

    pltpu.sync_copy(ia_hbm, pia_v)
    pltpu.sync_copy(ib_hbm, pib_v)
    pltpu.sync_copy(ll_hbm, lg_v)

    @pl.loop(0, NGC)
    def init_consts(gc):
        s = pl.ds(gc * L, L)
        j2 = (gc * (2 * L)) + 2 * lax.iota(jnp.int32, 16)
        la = plsc.load_gather(lg_v, [j2])
        lb = plsc.load_gather(lg_v, [j2 + 1])
        pia_v[s] = pia_v[s] | jnp.where(la > 0.0, SIGN, 0)
        pib_v[s] = pib_v[s] | jnp.where(lb > 0.0, SIGN, 0)

    zi = _zero_dep(pia_v[pl.ds(0, L)])

    @pl.loop(0, NBLK)
    def do_block(blk):
        row0 = row_base + blk * R
        pltpu.sync_copy(x_hbm.at[pl.ds(row0 * IN_DIM, R * IN_DIM)], xblk_v)
        zx = zi + _zero_dep(plsc.bitcast(xblk_v[pl.ds(0, L)], jnp.int32))

        def do_chunk(gc, c):
            s = pl.ds(gc * L + zx, L)
            pia = pia_v[s]
            pib = pib_v[s]
            ca = lax.shift_right_logical(pia, 31).astype(jnp.float32)
            cb = lax.shift_right_logical(pib, 31).astype(jnp.float32)
            sa = 1.0 - 2.0 * ca
            sb = 1.0 - 2.0 * cb
            ia = pia & MASK
            ib = pib & MASK
            for r in range(R):
                a = plsc.load_gather(xblk_v, [ia + (r * IN_DIM)])
                bb = plsc.load_gather(xblk_v, [ib + (r * IN_DIM)])
                oblk_v[pl.ds(r * OUT_DIM + gc * L, L)] = (
                    (ca + sa * a) * (cb + sb * bb))
            return c

        tot = plsc.parallel_loop(0, NGC, unroll=4,
                                 carry=jnp.int32(0))(do_chunk)
        off = pl.multiple_of(row0 * OUT_DIM + (tot & 0), 8)
        pltpu.sync_copy(oblk_v, out_hbm.at[pl.ds(off, R * OUT_DIM)])


@jax.jit
def kernel(x, negation_logits, idx_a, idx_b):
    mesh = plsc.VectorSubcoreMesh(core_axis_name="c", subcore_axis_name="s")
    f = pl.kernel(
        _sc_body,
        out_type=jax.ShapeDtypeStruct((BATCH * OUT_DIM,), jnp.float32),
        mesh=mesh,
        compiler_params=pltpu.CompilerParams(needs_layout_passes=False),
        scratch_types=[
            pltpu.VMEM((OUT_DIM,), jnp.int32),
            pltpu.VMEM((OUT_DIM,), jnp.int32),
            pltpu.VMEM((2 * OUT_DIM,), jnp.float32),
            pltpu.VMEM((R * IN_DIM,), jnp.float32),
            pltpu.VMEM((R * OUT_DIM,), jnp.float32),
        ],
    )
    out = f(x.reshape(-1), negation_logits.reshape(-1), idx_a, idx_b)
    return out.reshape(BATCH, OUT_DIM)
